# baseline (speedup 1.0000x reference)
import functools

import jax
import jax.numpy as jnp
from jax import lax
from jax.experimental import pallas as pl
from jax.experimental.pallas import tpu as pltpu
from jax.experimental.pallas import tpu_sc as plsc

_N = 10000
_D = 128
_E = 320000
_E_TAR = 8192

_NC = 2
_NS = 16
_NG = 3

_N_PAD = 10240
_CHUNK = 128
_NCHUNK = 80
_EPT = _NCHUNK * _CHUNK
_E_PAD = _NC * _NS * _EPT
_TROWS = _N_PAD // _NS

_GCHUNK = 64
_E_ALL = 2 * _E_TAR


def _prep_body(x_ref, deg_ref, x3_ref, degrev_ref):
    x = x_ref[...]
    deg = deg_ref[...]
    x3_ref[0] = x
    x3_ref[1] = x * lax.rsqrt(deg)
    x3_ref[2] = x * jnp.sqrt(deg)
    degrev_ref[...] = jnp.reciprocal(deg)


_prep = pl.pallas_call(
    _prep_body,
    out_shape=[
        jax.ShapeDtypeStruct((_NG, _N_PAD, _D), jnp.float32),
        jax.ShapeDtypeStruct((_N_PAD, 1), jnp.float32),
    ],
)


def _make_spmm():
    mesh = plsc.VectorSubcoreMesh(core_axis_name="c", subcore_axis_name="s")

    scratch = [
        pltpu.VMEM((_CHUNK,), jnp.int32),
        pltpu.VMEM((_CHUNK,), jnp.int32),
        pltpu.VMEM((_CHUNK, _D), jnp.float32),
        pltpu.VMEM((64, _D), jnp.float32),
        pltpu.VMEM_SHARED((_N_PAD, _D), jnp.float32),
        pltpu.SemaphoreType.DMA,
    ]

    def body(src, col, row, out, colv, rowv, gbuf, zbuf, acc, sem):
        cid = lax.axis_index("c")
        sid = lax.axis_index("s")
        ebase = (cid * _NS + sid) * _EPT
        zero = jnp.zeros((16,), jnp.float32)

        def zrow(r, carry):
            for v in range(_D // 16):
                zbuf[r, pl.ds(v * 16, 16)] = zero
            return carry

        lax.fori_loop(0, 64, zrow, 0)
        tb = sid * _TROWS

        for g in range(_NG):
            for i in range(_TROWS // 64):
                pltpu.sync_copy(zbuf, acc.at[pl.ds(tb + i * 64, 64)])
            plsc.subcore_barrier()

            def chunk(c, carry):
                b = ebase + c * _CHUNK
                pltpu.sync_copy(col.at[pl.ds(b, _CHUNK)], colv)
                pltpu.sync_copy(row.at[pl.ds(b, _CHUNK)], rowv)
                pltpu.async_copy(src.at[g].at[colv], gbuf, sem).wait()
                pltpu.sync_copy(gbuf, acc.at[rowv], add=True)
                return carry

            lax.fori_loop(0, _NCHUNK, chunk, 0)
            plsc.subcore_barrier()

            pltpu.sync_copy(acc.at[pl.ds(tb, _TROWS)],
                            out.at[g, cid, pl.ds(tb, _TROWS)])
            plsc.subcore_barrier()

    return functools.partial(
        pl.kernel,
        mesh=mesh,
        out_type=jax.ShapeDtypeStruct((_NG, _NC, _N_PAD, _D), jnp.float32),
        scratch_types=scratch,
    )(body)


_spmm = _make_spmm()


def _combine_body(p_ref, degrev_ref, sub_ref, out_ref, *, subtract):
    dr = degrev_ref[...]
    for g in range(_NG):
        h = (p_ref[g, 0] + p_ref[g, 1]) * dr
        if subtract:
            h = h - sub_ref[g]
        out_ref[g] = h


def _make_combine(subtract: bool):
    bn = 1024
    grid = (_N_PAD // bn,)
    return pl.pallas_call(
        functools.partial(_combine_body, subtract=subtract),
        grid=grid,
        in_specs=[
            pl.BlockSpec((_NG, _NC, bn, _D), lambda i: (0, 0, i, 0)),
            pl.BlockSpec((bn, 1), lambda i: (i, 0)),
            pl.BlockSpec((_NG, bn, _D), lambda i: (0, i, 0)),
        ],
        out_specs=pl.BlockSpec((_NG, bn, _D), lambda i: (0, i, 0)),
        out_shape=jax.ShapeDtypeStruct((_NG, _N_PAD, _D), jnp.float32),
    )


_combine = _make_combine(False)
_combine_sub = _make_combine(True)


def _make_edge_gather():
    mesh = plsc.VectorSubcoreMesh(core_axis_name="c", subcore_axis_name="s")

    scratch = [
        pltpu.VMEM((_GCHUNK,), jnp.int32),
    ] + [
        pltpu.VMEM((_GCHUNK, _D), jnp.float32) for _ in range(9)
    ] + [
        pltpu.SemaphoreType.DMA,
    ]

    def body(x3, h1, h2, eidx, out, ev, *rest):
        gbufs = rest[:9]
        sem = rest[9]
        cid = lax.axis_index("c")
        sid = lax.axis_index("s")
        wid = sid * _NC + cid
        per_w = _E_ALL // (_NC * _NS)

        def chunk(c, carry):
            b = wid * per_w + c * _GCHUNK
            pltpu.sync_copy(eidx.at[pl.ds(b, _GCHUNK)], ev)
            for a in range(9):
                src = (x3, h1, h2)[a // 3]
                pltpu.make_async_copy(src.at[a % 3].at[ev], gbufs[a],
                                      sem).start()
            for a in range(9):
                src = (x3, h1, h2)[a // 3]
                pltpu.make_async_copy(src.at[a % 3].at[ev], gbufs[a],
                                      sem).wait()
            for a in range(9):
                pltpu.sync_copy(
                    gbufs[a], out.at[pl.ds(b, _GCHUNK), pl.ds(a * _D, _D)])
            return carry

        lax.fori_loop(0, per_w // _GCHUNK, chunk, 0)

    return functools.partial(
        pl.kernel,
        mesh=mesh,
        out_type=jax.ShapeDtypeStruct((_E_ALL, 9 * _D), jnp.float32),
        scratch_types=scratch,
    )(body)


_edge_gather = _make_edge_gather()


def kernel(x, edge, adj_index, deg):
    x_pad = jnp.pad(x, ((0, _N_PAD - _N), (0, 0)))
    deg_pad = jnp.pad(deg, ((0, _N_PAD - _N), (0, 0)), constant_values=1.0)
    row = jnp.pad(adj_index[0], (0, _E_PAD - _E), constant_values=_N_PAD - 1)
    col = jnp.pad(adj_index[1], (0, _E_PAD - _E))

    x3, degrev = _prep(x_pad, deg_pad)
    zero_sub = jnp.zeros((_NG, _N_PAD, _D), jnp.float32)
    h1 = _combine(_spmm(x3, col, row), degrev, zero_sub)
    h2 = _combine_sub(_spmm(h1, col, row), degrev, x3)

    g = _edge_gather(x3, h1, h2, edge.reshape(_E_ALL))
    out = g.reshape(_E_ALL, 9, _D).transpose(0, 2, 1)
    return out.reshape(2, _E_TAR, _D, 9)


# speedup vs baseline: 1.3859x; 1.3859x over previous
import functools

import jax
import jax.numpy as jnp
from jax import lax
from jax.experimental import pallas as pl
from jax.experimental.pallas import tpu as pltpu
from jax.experimental.pallas import tpu_sc as plsc

_N = 10000
_D = 128
_E = 320000
_E_TAR = 8192

_NC = 2
_NS = 16
_NG = 3

_N_PAD = 10240
_CHUNK = 128
_NCHUNK = 79
_EPT = _NCHUNK * _CHUNK
_E_PAD = _NC * _NS * _EPT
_TROWS = _N_PAD // _NS

_GCHUNK = 64
_E_ALL = 2 * _E_TAR


def _prep_body(x_ref, deg_ref, x3_ref, degrev_ref):
    x = x_ref[...]
    deg = deg_ref[...]
    x3_ref[0] = x
    x3_ref[1] = x * lax.rsqrt(deg)
    x3_ref[2] = x * jnp.sqrt(deg)
    degrev_ref[...] = jnp.reciprocal(deg)


_prep = pl.pallas_call(
    _prep_body,
    out_shape=[
        jax.ShapeDtypeStruct((_NG, _N_PAD, _D), jnp.float32),
        jax.ShapeDtypeStruct((_N_PAD, 1), jnp.float32),
    ],
)


def _make_spmm():
    mesh = plsc.VectorSubcoreMesh(core_axis_name="c", subcore_axis_name="s")

    scratch = [
        pltpu.VMEM((_CHUNK,), jnp.int32),
        pltpu.VMEM((_CHUNK,), jnp.int32),
        pltpu.VMEM((_CHUNK, _D), jnp.float32),
        pltpu.VMEM((64, _D), jnp.float32),
        pltpu.VMEM_SHARED((_N_PAD, _D), jnp.float32),
        pltpu.SemaphoreType.DMA,
    ]

    def body(src, col, row, out, colv, rowv, gbuf, zbuf, acc, sem):
        cid = lax.axis_index("c")
        sid = lax.axis_index("s")
        ebase = (cid * _NS + sid) * _EPT
        zero = jnp.zeros((16,), jnp.float32)

        def zrow(r, carry):
            for v in range(_D // 16):
                zbuf[r, pl.ds(v * 16, 16)] = zero
            return carry

        lax.fori_loop(0, 64, zrow, 0)
        tb = sid * _TROWS

        for g in range(_NG):
            def zpiece(i, carry):
                pltpu.sync_copy(zbuf, acc.at[pl.ds(tb + i * 64, 64)])
                return carry

            lax.fori_loop(0, _TROWS // 64, zpiece, 0)
            plsc.subcore_barrier()

            def chunk(c, carry):
                b = ebase + c * _CHUNK
                pltpu.sync_copy(col.at[pl.ds(b, _CHUNK)], colv)
                pltpu.sync_copy(row.at[pl.ds(b, _CHUNK)], rowv)
                pltpu.async_copy(src.at[g].at[colv], gbuf, sem).wait()
                pltpu.sync_copy(gbuf, acc.at[rowv], add=True)
                return carry

            lax.fori_loop(0, _NCHUNK, chunk, 0)
            plsc.subcore_barrier()

            pltpu.sync_copy(acc.at[pl.ds(tb, _TROWS)],
                            out.at[g, cid, pl.ds(tb, _TROWS)])
            plsc.subcore_barrier()


    return functools.partial(
        pl.kernel,
        mesh=mesh,
        out_type=jax.ShapeDtypeStruct((_NG, _NC, _N_PAD, _D), jnp.float32),
        scratch_types=scratch,
    )(body)


_spmm = _make_spmm()


def _combine_body(p_ref, degrev_ref, sub_ref, out_ref, *, subtract):
    dr = degrev_ref[...]
    for g in range(_NG):
        h = (p_ref[g, 0] + p_ref[g, 1]) * dr
        if subtract:
            h = h - sub_ref[g]
        out_ref[g] = h


def _make_combine(subtract: bool):
    bn = 1024
    grid = (_N_PAD // bn,)
    return pl.pallas_call(
        functools.partial(_combine_body, subtract=subtract),
        grid=grid,
        in_specs=[
            pl.BlockSpec((_NG, _NC, bn, _D), lambda i: (0, 0, i, 0)),
            pl.BlockSpec((bn, 1), lambda i: (i, 0)),
            pl.BlockSpec((_NG, bn, _D), lambda i: (0, i, 0)),
        ],
        out_specs=pl.BlockSpec((_NG, bn, _D), lambda i: (0, i, 0)),
        out_shape=jax.ShapeDtypeStruct((_NG, _N_PAD, _D), jnp.float32),
    )


_combine = _make_combine(False)
_combine_sub = _make_combine(True)


def _make_edge_gather():
    mesh = plsc.VectorSubcoreMesh(core_axis_name="c", subcore_axis_name="s")

    scratch = [
        pltpu.VMEM((_GCHUNK,), jnp.int32),
    ] + [
        pltpu.VMEM((_GCHUNK, _D), jnp.float32) for _ in range(9)
    ] + [
        pltpu.SemaphoreType.DMA,
    ]

    def body(x3, h1, h2, eidx, out, ev, *rest):
        gbufs = rest[:9]
        sem = rest[9]
        cid = lax.axis_index("c")
        sid = lax.axis_index("s")
        wid = sid * _NC + cid
        per_w = _E_ALL // (_NC * _NS)

        def chunk(c, carry):
            b = wid * per_w + c * _GCHUNK
            pltpu.sync_copy(eidx.at[pl.ds(b, _GCHUNK)], ev)
            for a in range(9):
                src = (x3, h1, h2)[a // 3]
                pltpu.async_copy(src.at[a % 3].at[ev], gbufs[a], sem).wait()
            for a in range(9):
                pltpu.sync_copy(
                    gbufs[a], out.at[pl.ds(b, _GCHUNK), pl.ds(a * _D, _D)])
            return carry

        lax.fori_loop(0, per_w // _GCHUNK, chunk, 0)

    return functools.partial(
        pl.kernel,
        mesh=mesh,
        out_type=jax.ShapeDtypeStruct((_E_ALL, 9 * _D), jnp.float32),
        scratch_types=scratch,
    )(body)


_edge_gather = _make_edge_gather()


def kernel(x, edge, adj_index, deg):
    x_pad = jnp.pad(x, ((0, _N_PAD - _N), (0, 0)))
    deg_pad = jnp.pad(deg, ((0, _N_PAD - _N), (0, 0)), constant_values=1.0)
    row = jnp.pad(adj_index[0], (0, _E_PAD - _E), constant_values=_N_PAD - 1)
    col = jnp.pad(adj_index[1], (0, _E_PAD - _E))

    x3, degrev = _prep(x_pad, deg_pad)
    zero_sub = jnp.zeros((_NG, _N_PAD, _D), jnp.float32)
    h1 = _combine(_spmm(x3, col, row), degrev, zero_sub)
    h2 = _combine_sub(_spmm(h1, col, row), degrev, x3)

    g = _edge_gather(x3, h1, h2, edge.reshape(_E_ALL))
    out = g.reshape(_E_ALL, 9, _D).transpose(0, 2, 1)
    return out.reshape(2, _E_TAR, _D, 9)


# speedup vs baseline: 2.0896x; 1.5078x over previous
import functools

import jax
import jax.numpy as jnp
from jax import lax
from jax.experimental import pallas as pl
from jax.experimental.pallas import tpu as pltpu
from jax.experimental.pallas import tpu_sc as plsc

_N = 10000
_D = 128
_E = 320000
_E_TAR = 8192

_NC = 2
_NS = 16
_NG = 3

_N_PAD = 10240
_CHUNK = 128
_NCHUNK = 79
_EPT = _NCHUNK * _CHUNK
_E_PAD = _NC * _NS * _EPT
_TROWS = _N_PAD // _NS

_GCHUNK = 64
_E_ALL = 2 * _E_TAR


def _prep_body(x_ref, deg_ref, x3_ref, degrev_ref):
    x = x_ref[...]
    deg = deg_ref[...]
    x3_ref[0] = x
    x3_ref[1] = x * lax.rsqrt(deg)
    x3_ref[2] = x * jnp.sqrt(deg)
    degrev_ref[...] = jnp.reciprocal(deg)


_prep = pl.pallas_call(
    _prep_body,
    out_shape=[
        jax.ShapeDtypeStruct((_NG, _N_PAD, _D), jnp.float32),
        jax.ShapeDtypeStruct((_N_PAD, 1), jnp.float32),
    ],
)


def _make_spmm():
    mesh = plsc.VectorSubcoreMesh(core_axis_name="c", subcore_axis_name="s")

    scratch = [
        pltpu.VMEM((_CHUNK,), jnp.int32),
        pltpu.VMEM((_CHUNK,), jnp.int32),
        pltpu.VMEM((_CHUNK, _D), jnp.float32),
        pltpu.VMEM((64, _D), jnp.float32),
        pltpu.VMEM_SHARED((_N_PAD, _D), jnp.float32),
        pltpu.SemaphoreType.DMA,
    ]

    def body(src, col, row, out, colv, rowv, gbuf, zbuf, acc, sem):
        cid = lax.axis_index("c")
        sid = lax.axis_index("s")
        ebase = (cid * _NS + sid) * _EPT
        zero = jnp.zeros((16,), jnp.float32)

        def zrow(r, carry):
            for v in range(_D // 16):
                zbuf[r, pl.ds(v * 16, 16)] = zero
            return carry

        lax.fori_loop(0, 64, zrow, 0)
        tb = sid * _TROWS

        for g in range(_NG):
            def zpiece(i, carry):
                pltpu.sync_copy(zbuf, acc.at[pl.ds(tb + i * 64, 64)])
                return carry

            lax.fori_loop(0, _TROWS // 64, zpiece, 0)
            plsc.subcore_barrier()

            def chunk(c, carry):
                b = ebase + c * _CHUNK
                pltpu.sync_copy(col.at[pl.ds(b, _CHUNK)], colv)
                pltpu.sync_copy(row.at[pl.ds(b, _CHUNK)], rowv)
                pltpu.async_copy(src.at[g].at[colv], gbuf, sem).wait()
                pltpu.sync_copy(gbuf, acc.at[rowv], add=True)
                return carry

            lax.fori_loop(0, _NCHUNK, chunk, 0)
            plsc.subcore_barrier()

            pltpu.sync_copy(acc.at[pl.ds(tb, _TROWS)],
                            out.at[g, cid, pl.ds(tb, _TROWS)])
            plsc.subcore_barrier()


    return functools.partial(
        pl.kernel,
        mesh=mesh,
        out_type=jax.ShapeDtypeStruct((_NG, _NC, _N_PAD, _D), jnp.float32),
        scratch_types=scratch,
    )(body)


_spmm = _make_spmm()


def _combine_body(p_ref, degrev_ref, sub_ref, out_ref, *, subtract):
    dr = degrev_ref[...]
    for g in range(_NG):
        h = (p_ref[g, 0] + p_ref[g, 1]) * dr
        if subtract:
            h = h - sub_ref[g]
        out_ref[g] = h


def _make_combine(subtract: bool):
    bn = 1024
    grid = (_N_PAD // bn,)
    return pl.pallas_call(
        functools.partial(_combine_body, subtract=subtract),
        grid=grid,
        in_specs=[
            pl.BlockSpec((_NG, _NC, bn, _D), lambda i: (0, 0, i, 0)),
            pl.BlockSpec((bn, 1), lambda i: (i, 0)),
            pl.BlockSpec((_NG, bn, _D), lambda i: (0, i, 0)),
        ],
        out_specs=pl.BlockSpec((_NG, bn, _D), lambda i: (0, i, 0)),
        out_shape=jax.ShapeDtypeStruct((_NG, _N_PAD, _D), jnp.float32),
    )


_combine = _make_combine(False)
_combine_sub = _make_combine(True)


def _make_edge_gather():
    mesh = plsc.VectorSubcoreMesh(core_axis_name="c", subcore_axis_name="s")

    scratch = [
        pltpu.VMEM((_GCHUNK,), jnp.int32),
    ] + [
        pltpu.VMEM((_GCHUNK, _D), jnp.float32) for _ in range(9)
    ] + [
        pltpu.SemaphoreType.DMA,
    ]

    def body(x3, h1, h2, eidx, out, ev, *rest):
        gbufs = rest[:9]
        sem = rest[9]
        cid = lax.axis_index("c")
        sid = lax.axis_index("s")
        wid = sid * _NC + cid
        per_w = _E_ALL // (_NC * _NS)

        def chunk(c, carry):
            b = wid * per_w + c * _GCHUNK
            pltpu.sync_copy(eidx.at[pl.ds(b, _GCHUNK)], ev)
            for a in range(9):
                src = (x3, h1, h2)[a // 3]
                pltpu.async_copy(src.at[a % 3].at[ev], gbufs[a], sem).wait()
            for a in range(9):
                pltpu.sync_copy(
                    gbufs[a], out.at[pl.ds(b, _GCHUNK), pl.ds(a * _D, _D)])
            return carry

        lax.fori_loop(0, per_w // _GCHUNK, chunk, 0)

    return functools.partial(
        pl.kernel,
        mesh=mesh,
        out_type=jax.ShapeDtypeStruct((_E_ALL, 9 * _D), jnp.float32),
        scratch_types=scratch,
    )(body)


_edge_gather = _make_edge_gather()


def kernel(x, edge, adj_index, deg):
    x_pad = jnp.pad(x, ((0, _N_PAD - _N), (0, 0)))
    deg_pad = jnp.pad(deg, ((0, _N_PAD - _N), (0, 0)), constant_values=1.0)
    padidx = _N + (jnp.arange(_E_PAD - _E, dtype=jnp.int32) % (_N_PAD - _N))
    row = jnp.concatenate([adj_index[0], padidx])
    col = jnp.concatenate([adj_index[1], padidx])

    x3, degrev = _prep(x_pad, deg_pad)
    zero_sub = jnp.zeros((_NG, _N_PAD, _D), jnp.float32)
    h1 = _combine(_spmm(x3, col, row), degrev, zero_sub)
    h2 = _combine_sub(_spmm(h1, col, row), degrev, x3)

    g = _edge_gather(x3, h1, h2, edge.reshape(_E_ALL))
    out = g.reshape(_E_ALL, 9, _D).transpose(0, 2, 1)
    return out.reshape(2, _E_TAR, _D, 9)


# speedup vs baseline: 2.1480x; 1.0279x over previous
import functools

import jax
import jax.numpy as jnp
from jax import lax
from jax.experimental import pallas as pl
from jax.experimental.pallas import tpu as pltpu
from jax.experimental.pallas import tpu_sc as plsc

_N = 10000
_D = 128
_E = 320000
_E_TAR = 8192

_NC = 2
_NS = 16
_NG = 3

_N_PAD = 10240
_CHUNK = 128
_NCHUNK = 79
_EPT = _NCHUNK * _CHUNK
_E_PAD = _NC * _NS * _EPT
_TROWS = _N_PAD // _NS

_GCHUNK = 64
_E_ALL = 2 * _E_TAR


def _prep_body(x_ref, deg_ref, x3_ref, degrev_ref):
    x = x_ref[...]
    deg = deg_ref[...]
    x3_ref[0] = x
    x3_ref[1] = x * lax.rsqrt(deg)
    x3_ref[2] = x * jnp.sqrt(deg)
    degrev_ref[...] = jnp.reciprocal(deg)


_prep = pl.pallas_call(
    _prep_body,
    out_shape=[
        jax.ShapeDtypeStruct((_NG, _N_PAD, _D), jnp.float32),
        jax.ShapeDtypeStruct((_N_PAD, 1), jnp.float32),
    ],
)


def _make_spmm():
    mesh = plsc.VectorSubcoreMesh(core_axis_name="c", subcore_axis_name="s")

    scratch = [
        pltpu.VMEM((_CHUNK,), jnp.int32),
        pltpu.VMEM((_CHUNK,), jnp.int32),
        pltpu.VMEM((_CHUNK, _D), jnp.float32),
        pltpu.VMEM((64, _D), jnp.float32),
        pltpu.VMEM_SHARED((_N_PAD, _D), jnp.float32),
        pltpu.SemaphoreType.DMA,
    ]

    def body(src, col, row, out, colv, rowv, gbuf, zbuf, acc, sem):
        cid = lax.axis_index("c")
        sid = lax.axis_index("s")
        ebase = (cid * _NS + sid) * _EPT
        zero = jnp.zeros((16,), jnp.float32)

        def zrow(r, carry):
            for v in range(_D // 16):
                zbuf[r, pl.ds(v * 16, 16)] = zero
            return carry

        lax.fori_loop(0, 64, zrow, 0)
        tb = sid * _TROWS

        for g in range(_NG):
            def zpiece(i, carry):
                pltpu.sync_copy(zbuf, acc.at[pl.ds(tb + i * 64, 64)])
                return carry

            lax.fori_loop(0, _TROWS // 64, zpiece, 0)
            plsc.subcore_barrier()

            def chunk(c, carry):
                b = ebase + c * _CHUNK
                pltpu.sync_copy(col.at[pl.ds(b, _CHUNK)], colv)
                pltpu.sync_copy(row.at[pl.ds(b, _CHUNK)], rowv)
                pltpu.async_copy(src.at[g].at[colv], gbuf, sem).wait()
                pltpu.sync_copy(gbuf, acc.at[rowv], add=True)
                return carry

            lax.fori_loop(0, _NCHUNK, chunk, 0)
            plsc.subcore_barrier()

            pltpu.sync_copy(acc.at[pl.ds(tb, _TROWS)],
                            out.at[g, cid, pl.ds(tb, _TROWS)])
            plsc.subcore_barrier()


    return functools.partial(
        pl.kernel,
        mesh=mesh,
        out_type=jax.ShapeDtypeStruct((_NG, _NC, _N_PAD, _D), jnp.float32),
        scratch_types=scratch,
    )(body)


_spmm = _make_spmm()


def _combine_body(p_ref, degrev_ref, sub_ref, out_ref, *, subtract):
    dr = degrev_ref[...]
    for g in range(_NG):
        h = (p_ref[g, 0] + p_ref[g, 1]) * dr
        if subtract:
            h = h - sub_ref[g]
        out_ref[g] = h


def _make_combine(subtract: bool):
    bn = 1024
    grid = (_N_PAD // bn,)
    return pl.pallas_call(
        functools.partial(_combine_body, subtract=subtract),
        grid=grid,
        in_specs=[
            pl.BlockSpec((_NG, _NC, bn, _D), lambda i: (0, 0, i, 0)),
            pl.BlockSpec((bn, 1), lambda i: (i, 0)),
            pl.BlockSpec((_NG, bn, _D), lambda i: (0, i, 0)),
        ],
        out_specs=pl.BlockSpec((_NG, bn, _D), lambda i: (0, i, 0)),
        out_shape=jax.ShapeDtypeStruct((_NG, _N_PAD, _D), jnp.float32),
    )


_combine = _make_combine(False)
_combine_sub = _make_combine(True)


def _make_edge_gather():
    mesh = plsc.VectorSubcoreMesh(core_axis_name="c", subcore_axis_name="s")

    scratch = [
        pltpu.VMEM((_GCHUNK,), jnp.int32),
    ] + [
        pltpu.VMEM((_GCHUNK, _D), jnp.float32) for _ in range(9)
    ] + [
        pltpu.SemaphoreType.DMA,
    ]

    def body(x3, h1, h2, eidx, out, ev, *rest):
        gbufs = rest[:9]
        sem = rest[9]
        cid = lax.axis_index("c")
        sid = lax.axis_index("s")
        wid = sid * _NC + cid
        per_w = _E_ALL // (_NC * _NS)

        def chunk(c, carry):
            b = wid * per_w + c * _GCHUNK
            pltpu.sync_copy(eidx.at[pl.ds(b, _GCHUNK)], ev)
            for a in range(9):
                src = (x3, h1, h2)[a // 3]
                pltpu.make_async_copy(src.at[a % 3].at[ev], gbufs[a],
                                      sem).start()
            for a in range(9):
                src = (x3, h1, h2)[a // 3]
                pltpu.make_async_copy(src.at[a % 3].at[ev], gbufs[a],
                                      sem).wait()
            for a in range(9):
                pltpu.sync_copy(
                    gbufs[a], out.at[pl.ds(b, _GCHUNK), pl.ds(a * _D, _D)])
            return carry

        lax.fori_loop(0, per_w // _GCHUNK, chunk, 0)

    return functools.partial(
        pl.kernel,
        mesh=mesh,
        out_type=jax.ShapeDtypeStruct((_E_ALL, 9 * _D), jnp.float32),
        scratch_types=scratch,
    )(body)


_edge_gather = _make_edge_gather()


def kernel(x, edge, adj_index, deg):
    x_pad = jnp.pad(x, ((0, _N_PAD - _N), (0, 0)))
    deg_pad = jnp.pad(deg, ((0, _N_PAD - _N), (0, 0)), constant_values=1.0)
    padidx = _N + (jnp.arange(_E_PAD - _E, dtype=jnp.int32) % (_N_PAD - _N))
    row = jnp.concatenate([adj_index[0], padidx])
    col = jnp.concatenate([adj_index[1], padidx])

    x3, degrev = _prep(x_pad, deg_pad)
    zero_sub = jnp.zeros((_NG, _N_PAD, _D), jnp.float32)
    h1 = _combine(_spmm(x3, col, row), degrev, zero_sub)
    h2 = _combine_sub(_spmm(h1, col, row), degrev, x3)

    g = _edge_gather(x3, h1, h2, edge.reshape(_E_ALL))
    out = g.reshape(_E_ALL, 9, _D).transpose(0, 2, 1)
    return out.reshape(2, _E_TAR, _D, 9)
